# baseline (speedup 1.0000x reference)
import functools

import jax
import jax.numpy as jnp
from jax import lax
from jax.experimental import pallas as pl
from jax.experimental.pallas import tpu as pltpu
from jax.experimental.pallas import tpu_sc as plsc

_NC = 2
_NS = 16
_NW = _NC * _NS
_L = 16


@functools.lru_cache(maxsize=None)
def _build(B, D):
    b_per_w = B // _NW
    n_groups = b_per_w // _L
    mesh = plsc.VectorSubcoreMesh(core_axis_name="c", subcore_axis_name="s")

    def body(ue_hbm, ie_hbm, ub_hbm, ib_hbm, uidx_hbm, iidx_hbm, out_hbm,
             uidx_v, iidx_v, urows_v, irows_v, ub_v, ib_v, out_v, sem):
        wid = lax.axis_index("s") * _NC + lax.axis_index("c")
        base = wid * b_per_w

        pltpu.sync_copy(uidx_hbm.at[pl.ds(base, b_per_w)], uidx_v)
        pltpu.sync_copy(iidx_hbm.at[pl.ds(base, b_per_w)], iidx_v)

        cps = [
            pltpu.async_copy(ue_hbm.at[uidx_v], urows_v, sem),
            pltpu.async_copy(ie_hbm.at[iidx_v], irows_v, sem),
            pltpu.async_copy(ub_hbm.at[uidx_v], ub_v, sem),
            pltpu.async_copy(ib_hbm.at[iidx_v], ib_v, sem),
        ]
        for cp in cps:
            cp.wait()

        lane = lax.iota(jnp.int32, 16)

        def group(g, carry):
            rbase = g * _L
            rows = rbase + lane
            acc = ub_v[pl.ds(rbase, _L)] + ib_v[pl.ds(rbase, _L)]
            for d in range(D):
                col = jnp.full((_L,), d, jnp.int32)
                acc += (plsc.load_gather(urows_v, [rows, col])
                        * plsc.load_gather(irows_v, [rows, col]))
            out_v[pl.ds(rbase, _L)] = acc
            return carry

        lax.fori_loop(0, n_groups, group, 0)

        pltpu.sync_copy(out_v, out_hbm.at[pl.ds(base, b_per_w)])

    return pl.kernel(
        body,
        out_type=jax.ShapeDtypeStruct((B,), jnp.float32),
        mesh=mesh,
        scratch_types=[
            pltpu.VMEM((b_per_w,), jnp.int32),
            pltpu.VMEM((b_per_w,), jnp.int32),
            pltpu.VMEM((b_per_w, D), jnp.float32),
            pltpu.VMEM((b_per_w, D), jnp.float32),
            pltpu.VMEM((b_per_w,), jnp.float32),
            pltpu.VMEM((b_per_w,), jnp.float32),
            pltpu.VMEM((b_per_w,), jnp.float32),
            pltpu.SemaphoreType.DMA,
        ],
        compiler_params=pltpu.CompilerParams(
            needs_layout_passes=False, use_tc_tiling_on_sc=False),
    )


def kernel(user, item, user_embedding, item_embedding, user_biases, item_biases):
    B = user.shape[0]
    D = user_embedding.shape[1]
    fn = _build(B, D)
    return fn(user_embedding, item_embedding,
              user_biases.reshape(-1), item_biases.reshape(-1),
              user.astype(jnp.int32), item.astype(jnp.int32))
